# baseline (speedup 1.0000x reference)
import functools

import jax
import jax.numpy as jnp
from jax import lax
from jax.experimental import pallas as pl
from jax.experimental.pallas import tpu as pltpu
from jax.experimental.pallas import tpu_sc as plsc

NC = 2
NS = 16
NW = NC * NS
D = 256
CW = 16
K = 64


@functools.lru_cache(maxsize=None)
def _make_segsum(P, Ep, n_keep, n_acc, spmem_table=False):
    per_w = Ep // NW
    assert per_w * NW == Ep and per_w % K == 0 and per_w % 16 == 0
    ncap = per_w + 2 * K
    assert n_acc >= n_keep + 16 and n_acc % 8 == 0
    rps = (-(-n_keep // NS) + 7) // 8 * 8
    n_full = n_keep // rps
    rem = n_keep - n_full * rps
    zps = (-(-n_acc // NS) + 7) // 8 * 8
    z_full = n_acc // zps
    z_rem = n_acc - z_full * zps
    B0 = 16
    assert rps % B0 == 0 and rem % B0 == 0 and zps % B0 == 0 and z_rem % B0 == 0
    mesh = plsc.VectorSubcoreMesh(core_axis_name="c", subcore_axis_name="s",
                                  num_cores=NC, num_subcores=NS)

    @functools.partial(
        pl.kernel,
        out_type=(jax.ShapeDtypeStruct((NC, n_keep, D), jnp.float32),
                  jax.ShapeDtypeStruct((NC, n_keep, CW), jnp.float32)),
        mesh=mesh,
        scratch_types=[
            pltpu.VMEM_SHARED((n_acc, D), jnp.float32),
            pltpu.VMEM_SHARED((n_acc, CW), jnp.float32),
            pltpu.VMEM((per_w,), jnp.int32),
            pltpu.VMEM((per_w,), jnp.int32),
            pltpu.VMEM((ncap,), jnp.int32),
            pltpu.VMEM((ncap,), jnp.int32),
            pltpu.VMEM((2, K, D), jnp.float32),
            pltpu.VMEM((K, CW), jnp.float32),
            pltpu.VMEM((B0, D), jnp.float32),
            pltpu.VMEM((B0, CW), jnp.float32),
            pltpu.SemaphoreType.DMA,
            pltpu.SemaphoreType.DMA,
        ] + ([pltpu.VMEM_SHARED((P, D), jnp.float32)] if spmem_table else []),
        compiler_params=pltpu.CompilerParams(use_tc_tiling_on_sc=False,
                                             needs_layout_passes=False),
    )
    def segsum(table_hbm, src_hbm, dst_hbm, zrow_hbm, zcnt_hbm, onesrow_hbm,
               parts_hbm, cparts_hbm,
               acc_sh, cacc_sh, src_v, dst_v, srcc_v, dstc_v, rows_v, ones_v,
               buf_v, bufc_v, sem0, sem1, *maybe_table_sh):
        c = lax.axis_index("c")
        s = lax.axis_index("s")
        wid = s * NC + c

        if spmem_table:
            table_sh = maybe_table_sh[0]
            tps = P // NS
            toff = pl.multiple_of(s * tps, 8)
            for j in range(tps // B0):
                pltpu.sync_copy(table_hbm.at[pl.ds(toff + j * B0, B0)], buf_v)
                pltpu.sync_copy(buf_v, table_sh.at[pl.ds(toff + j * B0, B0)])
            gather_src = table_sh
        else:
            gather_src = table_hbm

        pltpu.sync_copy(src_hbm.at[pl.ds(wid * per_w, per_w)], src_v)
        pltpu.sync_copy(dst_hbm.at[pl.ds(wid * per_w, per_w)], dst_v)
        pltpu.sync_copy(zrow_hbm, buf_v)
        pltpu.sync_copy(zcnt_hbm, bufc_v)
        pltpu.sync_copy(onesrow_hbm, ones_v)

        zoff = pl.multiple_of(s * zps, 8)

        @pl.when(s < z_full)
        def _zero_full():
            for j in range(zps // B0):
                pltpu.sync_copy(buf_v, acc_sh.at[pl.ds(zoff + j * B0, B0)])
            for j in range(zps // B0):
                pltpu.sync_copy(bufc_v, cacc_sh.at[pl.ds(zoff + j * B0, B0)])
        if z_rem:
            @pl.when(s == z_full)
            def _zero_rem():
                for j in range(z_rem // B0):
                    pltpu.sync_copy(
                        buf_v, acc_sh.at[pl.ds(z_full * zps + j * B0, B0)])
                for j in range(z_rem // B0):
                    pltpu.sync_copy(
                        bufc_v, cacc_sh.at[pl.ds(z_full * zps + j * B0, B0)])

        lane = lax.iota(jnp.int32, 16)

        def fill(g, carry):
            srcc_v[pl.ds(g * 16, 16)] = lane
            dstc_v[pl.ds(g * 16, 16)] = lane + n_keep
            return carry

        lax.fori_loop(0, ncap // 16, fill, 0)

        def compact(g, off):
            sv = src_v[pl.ds(g * 16, 16)]
            dv = dst_v[pl.ds(g * 16, 16)]
            m = dv < n_keep
            packed = jnp.bitwise_or(jnp.left_shift(sv, 12), dv)
            _, pv = plsc.sort_key_val(dv, packed)
            nk = plsc.all_reduce_population_count(m)
            keep = lane < nk
            pos = off + lane
            plsc.store_scatter(srcc_v, [pos], jnp.right_shift(pv, 12), mask=keep)
            plsc.store_scatter(dstc_v, [pos], jnp.bitwise_and(pv, 4095), mask=keep)
            return off + nk

        offv = lax.fori_loop(0, per_w // 16, compact,
                             jnp.zeros((16,), jnp.int32))
        n_edges = jnp.max(offv)
        n_pairs = (n_edges + 2 * K - 1) // (2 * K)

        plsc.subcore_barrier()

        sems = (sem0, sem1)

        def gather(ci, b):
            pltpu.async_copy(gather_src.at[srcc_v.at[pl.ds(ci * K, K)]],
                             rows_v.at[b], sems[b])

        def gwait(ci, b):
            pltpu.make_async_copy(gather_src.at[srcc_v.at[pl.ds(ci * K, K)]],
                                  rows_v.at[b], sems[b]).wait()

        def scatter(ci, b):
            idx = dstc_v.at[pl.ds(ci * K, K)]
            pltpu.sync_copy(rows_v.at[b], acc_sh.at[idx], add=True)
            pltpu.sync_copy(ones_v, cacc_sh.at[idx], add=True)

        @pl.when(n_pairs > 0)
        def _prologue():
            gather(0, 0)

        def pair(i, carry):
            c0 = i * 2
            gwait(c0, 0)
            gather(c0 + 1, 1)
            scatter(c0, 0)
            gwait(c0 + 1, 1)

            @pl.when(i < n_pairs - 1)
            def _next():
                gather(c0 + 2, 0)
            scatter(c0 + 1, 1)
            return carry

        lax.fori_loop(0, n_pairs, pair, 0)
        plsc.subcore_barrier()

        row_off = pl.multiple_of(s * rps, 8)

        @pl.when(s < n_full)
        def _out_full():
            for j in range(rps // B0):
                pltpu.sync_copy(acc_sh.at[pl.ds(row_off + j * B0, B0)], buf_v)
                pltpu.sync_copy(buf_v,
                                parts_hbm.at[c, pl.ds(row_off + j * B0, B0)])
            for j in range(rps // B0):
                pltpu.sync_copy(cacc_sh.at[pl.ds(row_off + j * B0, B0)], bufc_v)
                pltpu.sync_copy(bufc_v,
                                cparts_hbm.at[c, pl.ds(row_off + j * B0, B0)])
        if rem:
            @pl.when(s == n_full)
            def _out_rem():
                for j in range(rem // B0):
                    off_r = n_full * rps + j * B0
                    pltpu.sync_copy(acc_sh.at[pl.ds(off_r, B0)], buf_v)
                    pltpu.sync_copy(buf_v, parts_hbm.at[c, pl.ds(off_r, B0)])
                for j in range(rem // B0):
                    off_r = n_full * rps + j * B0
                    pltpu.sync_copy(cacc_sh.at[pl.ds(off_r, B0)], bufc_v)
                    pltpu.sync_copy(bufc_v, cparts_hbm.at[c, pl.ds(off_r, B0)])

    return segsum


def _pad_edges(edge_index, E, Ep, dump):
    pad = Ep - E
    spread = jnp.arange(pad, dtype=jnp.int32) % K
    src = jnp.concatenate([edge_index[0], spread])
    dst = jnp.concatenate([edge_index[1], dump + spread])
    return src, dst


def _mean(parts_ref, cnt_ref):
    s = parts_ref[0] + parts_ref[1]
    cnt = cnt_ref[0, :, :1] + cnt_ref[1, :, :1]
    return s / jnp.maximum(cnt, 1.0)


def _t12_body(parts_ref, cnt_ref, x_ref, wl1_ref, wr1_ref, b_ref,
              wl2_ref, wr2_ref, z_ref, r2_ref):
    acc = jnp.dot(_mean(parts_ref, cnt_ref), wl1_ref[...],
                  preferred_element_type=jnp.float32)
    acc = acc + jnp.dot(x_ref[...], wr1_ref[...],
                        preferred_element_type=jnp.float32)
    h = jnp.maximum(acc + b_ref[...], 0.0)
    z_ref[...] = jnp.dot(h, wl2_ref[...], preferred_element_type=jnp.float32)
    r2_ref[...] = jnp.dot(h, wr2_ref[...], preferred_element_type=jnp.float32)


def _t12(parts, cnt, x1k, W_l1, W_r1, b1, W_l2, W_r2):
    M, H = 1024, 1024
    BM = 512
    return pl.pallas_call(
        _t12_body,
        grid=(M // BM,),
        in_specs=[
            pl.BlockSpec((NC, BM, D), lambda i: (0, i, 0)),
            pl.BlockSpec((NC, BM, CW), lambda i: (0, i, 0)),
            pl.BlockSpec((BM, D), lambda i: (i, 0)),
            pl.BlockSpec((D, H), lambda i: (0, 0)),
            pl.BlockSpec((D, H), lambda i: (0, 0)),
            pl.BlockSpec((1, H), lambda i: (0, 0)),
            pl.BlockSpec((H, D), lambda i: (0, 0)),
            pl.BlockSpec((H, D), lambda i: (0, 0)),
        ],
        out_specs=[
            pl.BlockSpec((BM, D), lambda i: (i, 0)),
            pl.BlockSpec((BM, D), lambda i: (i, 0)),
        ],
        out_shape=[
            jax.ShapeDtypeStruct((M, D), jnp.float32),
            jax.ShapeDtypeStruct((M, D), jnp.float32),
        ],
    )(parts, cnt, x1k, W_l1, W_r1, b1.reshape(1, H), W_l2, W_r2)


def _t3_body(parts_ref, cnt_ref, r2_ref, b_ref, out_ref):
    o = _mean(parts_ref, cnt_ref) + r2_ref[...] + b_ref[...]
    m = jnp.max(o, axis=1, keepdims=True)
    e = jnp.exp(o - m)
    lse = jnp.log(jnp.sum(e, axis=1, keepdims=True))
    out_ref[...] = (o - m) - lse


def _t3(parts, cnt, r2, b):
    M = 1024
    return pl.pallas_call(
        _t3_body,
        grid=(1,),
        in_specs=[
            pl.BlockSpec((NC, M, D), lambda i: (0, 0, 0)),
            pl.BlockSpec((NC, M, CW), lambda i: (0, 0, 0)),
            pl.BlockSpec((M, D), lambda i: (0, 0)),
            pl.BlockSpec((1, D), lambda i: (0, 0)),
        ],
        out_specs=pl.BlockSpec((M, D), lambda i: (0, 0)),
        out_shape=jax.ShapeDtypeStruct((M, D), jnp.float32),
    )(parts, cnt, r2, b.reshape(1, D))


def kernel(x, edge_index1, edge_index2, W_l1, W_r1, b1, W_l2, W_r2, b2,
           n_dst1, n_dst2):
    off1 = n_dst1 - 4000
    x4k = lax.dynamic_slice_in_dim(x, off1, 4000, axis=0)
    x1k = x4k[:1024]
    zrow = jnp.zeros((16, D), jnp.float32)
    zcnt = jnp.zeros((16, CW), jnp.float32)
    onesrow = jnp.ones((K, CW), jnp.float32)

    src1, dst1 = _pad_edges(edge_index1, 160000, 163840, 4000)
    parts1, cnt1 = _make_segsum(P=4000, Ep=163840, n_keep=1024, n_acc=4096)(
        x4k, src1, dst1, zrow, zcnt, onesrow)
    z, r2 = _t12(parts1, cnt1, x1k, W_l1, W_r1, b1, W_l2, W_r2)

    src2, dst2 = _pad_edges(edge_index2, 64000, 65536, 1024)
    parts2, cnt2 = _make_segsum(P=1024, Ep=65536, n_keep=1024, n_acc=1280,
                                spmem_table=True)(
        z, src2, dst2, zrow, zcnt, onesrow)
    return _t3(parts2, cnt2, r2, b2)


# speedup vs baseline: 1.0819x; 1.0819x over previous
import functools

import jax
import jax.numpy as jnp
from jax import lax
from jax.experimental import pallas as pl
from jax.experimental.pallas import tpu as pltpu
from jax.experimental.pallas import tpu_sc as plsc

NC = 2
NS = 16
NW = NC * NS
D = 256
CW = 16
K = 64


@functools.lru_cache(maxsize=None)
def _make_segsum(P, Ep, n_keep, n_acc, spmem_table=False):
    per_w = Ep // NW
    assert per_w * NW == Ep and per_w % K == 0 and per_w % 16 == 0
    ncap = per_w + 2 * K
    assert n_acc >= n_keep + 16 and n_acc % 8 == 0
    rps = (-(-n_keep // NS) + 7) // 8 * 8
    n_full = n_keep // rps
    rem = n_keep - n_full * rps
    zps = (-(-n_acc // NS) + 7) // 8 * 8
    z_full = n_acc // zps
    z_rem = n_acc - z_full * zps
    B0 = 16
    assert rps % B0 == 0 and rem % B0 == 0 and zps % B0 == 0 and z_rem % B0 == 0
    mesh = plsc.VectorSubcoreMesh(core_axis_name="c", subcore_axis_name="s",
                                  num_cores=NC, num_subcores=NS)

    @functools.partial(
        pl.kernel,
        out_type=(jax.ShapeDtypeStruct((NC, n_keep, D), jnp.float32),
                  jax.ShapeDtypeStruct((NC, n_keep, CW), jnp.float32)),
        mesh=mesh,
        scratch_types=[
            pltpu.VMEM_SHARED((n_acc, D), jnp.float32),
            pltpu.VMEM_SHARED((n_acc, CW), jnp.float32),
            pltpu.VMEM((per_w,), jnp.int32),
            pltpu.VMEM((per_w,), jnp.int32),
            pltpu.VMEM((ncap,), jnp.int32),
            pltpu.VMEM((ncap,), jnp.int32),
            pltpu.VMEM((2, K, D), jnp.float32),
            pltpu.VMEM((K, CW), jnp.float32),
            pltpu.VMEM((B0, D), jnp.float32),
            pltpu.VMEM((B0, CW), jnp.float32),
            pltpu.SemaphoreType.DMA,
            pltpu.SemaphoreType.DMA,
        ] + ([pltpu.VMEM_SHARED((P, D), jnp.float32)] if spmem_table else []),
        compiler_params=pltpu.CompilerParams(use_tc_tiling_on_sc=False,
                                             needs_layout_passes=False),
    )
    def segsum(table_hbm, src_hbm, dst_hbm, zrow_hbm, zcnt_hbm, onesrow_hbm,
               parts_hbm, cparts_hbm,
               acc_sh, cacc_sh, src_v, dst_v, srcc_v, dstc_v, rows_v, ones_v,
               buf_v, bufc_v, sem0, sem1, *maybe_table_sh):
        c = lax.axis_index("c")
        s = lax.axis_index("s")
        wid = s * NC + c

        if spmem_table:
            table_sh = maybe_table_sh[0]
            tps = P // NS
            toff = pl.multiple_of(s * tps, 8)
            for j in range(tps // B0):
                pltpu.sync_copy(table_hbm.at[pl.ds(toff + j * B0, B0)], buf_v)
                pltpu.sync_copy(buf_v, table_sh.at[pl.ds(toff + j * B0, B0)])
            gather_src = table_sh
        else:
            gather_src = table_hbm

        pltpu.sync_copy(src_hbm.at[pl.ds(wid * per_w, per_w)], src_v)
        pltpu.sync_copy(dst_hbm.at[pl.ds(wid * per_w, per_w)], dst_v)
        pltpu.sync_copy(zrow_hbm, buf_v)
        pltpu.sync_copy(zcnt_hbm, bufc_v)
        pltpu.sync_copy(onesrow_hbm, ones_v)

        zoff = pl.multiple_of(s * zps, 8)

        @pl.when(s < z_full)
        def _zero_full():
            for j in range(zps // B0):
                pltpu.sync_copy(buf_v, acc_sh.at[pl.ds(zoff + j * B0, B0)])
            for j in range(zps // B0):
                pltpu.sync_copy(bufc_v, cacc_sh.at[pl.ds(zoff + j * B0, B0)])
        if z_rem:
            @pl.when(s == z_full)
            def _zero_rem():
                for j in range(z_rem // B0):
                    pltpu.sync_copy(
                        buf_v, acc_sh.at[pl.ds(z_full * zps + j * B0, B0)])
                for j in range(z_rem // B0):
                    pltpu.sync_copy(
                        bufc_v, cacc_sh.at[pl.ds(z_full * zps + j * B0, B0)])

        lane = lax.iota(jnp.int32, 16)

        def fill(g, carry):
            srcc_v[pl.ds(g * 16, 16)] = lane
            dstc_v[pl.ds(g * 16, 16)] = lane + n_keep
            return carry

        lax.fori_loop(0, ncap // 16, fill, 0)

        def compact(g, off):
            sv = src_v[pl.ds(g * 16, 16)]
            dv = dst_v[pl.ds(g * 16, 16)]
            m = dv < n_keep
            packed = jnp.bitwise_or(jnp.left_shift(sv, 12), dv)
            _, pv = plsc.sort_key_val(dv, packed)
            nk = plsc.all_reduce_population_count(m)
            keep = lane < nk
            pos = off + lane
            plsc.store_scatter(srcc_v, [pos], jnp.right_shift(pv, 12), mask=keep)
            plsc.store_scatter(dstc_v, [pos], jnp.bitwise_and(pv, 4095), mask=keep)
            return off + nk

        offv = lax.fori_loop(0, per_w // 16, compact,
                             jnp.zeros((16,), jnp.int32))
        n_edges = jnp.max(offv)
        n_pairs = (n_edges + 2 * K - 1) // (2 * K)

        plsc.subcore_barrier()

        sems = (sem0, sem1)

        def gather(ci, b):
            pltpu.async_copy(gather_src.at[srcc_v.at[pl.ds(ci * K, K)]],
                             rows_v.at[b], sems[b])

        def gwait(ci, b):
            pltpu.make_async_copy(gather_src.at[srcc_v.at[pl.ds(ci * K, K)]],
                                  rows_v.at[b], sems[b]).wait()

        def scatter(ci, b):
            idx = dstc_v.at[pl.ds(ci * K, K)]
            pltpu.sync_copy(rows_v.at[b], acc_sh.at[idx], add=True)
            pltpu.sync_copy(ones_v, cacc_sh.at[idx], add=True)

        @pl.when(n_pairs > 0)
        def _prologue():
            gather(0, 0)

        def pair(i, carry):
            c0 = i * 2
            gwait(c0, 0)
            gather(c0 + 1, 1)
            scatter(c0, 0)
            gwait(c0 + 1, 1)

            @pl.when(i < n_pairs - 1)
            def _next():
                gather(c0 + 2, 0)
            scatter(c0 + 1, 1)
            return carry

        lax.fori_loop(0, n_pairs, pair, 0)
        plsc.subcore_barrier()

        row_off = pl.multiple_of(s * rps, 8)

        @pl.when(s < n_full)
        def _out_full():
            for j in range(rps // B0):
                pltpu.sync_copy(acc_sh.at[pl.ds(row_off + j * B0, B0)], buf_v)
                pltpu.sync_copy(buf_v,
                                parts_hbm.at[c, pl.ds(row_off + j * B0, B0)])
            for j in range(rps // B0):
                pltpu.sync_copy(cacc_sh.at[pl.ds(row_off + j * B0, B0)], bufc_v)
                pltpu.sync_copy(bufc_v,
                                cparts_hbm.at[c, pl.ds(row_off + j * B0, B0)])
        if rem:
            @pl.when(s == n_full)
            def _out_rem():
                for j in range(rem // B0):
                    off_r = n_full * rps + j * B0
                    pltpu.sync_copy(acc_sh.at[pl.ds(off_r, B0)], buf_v)
                    pltpu.sync_copy(buf_v, parts_hbm.at[c, pl.ds(off_r, B0)])
                for j in range(rem // B0):
                    off_r = n_full * rps + j * B0
                    pltpu.sync_copy(cacc_sh.at[pl.ds(off_r, B0)], bufc_v)
                    pltpu.sync_copy(bufc_v, cparts_hbm.at[c, pl.ds(off_r, B0)])

    return segsum


def _pad_edges(edge_index, E, Ep, dump):
    pad = Ep - E
    spread = jnp.arange(pad, dtype=jnp.int32) % K
    src = jnp.concatenate([edge_index[0], spread])
    dst = jnp.concatenate([edge_index[1], dump + spread])
    return src, dst


def _mean(parts_ref, cnt_ref):
    s = parts_ref[0] + parts_ref[1]
    cnt = cnt_ref[0, :, :1] + cnt_ref[1, :, :1]
    return s / jnp.maximum(cnt, 1.0)


def _t12_body(parts_ref, cnt_ref, x_ref, wl1_ref, wr1_ref, b_ref,
              wl2_ref, wr2_ref, z_ref, r2_ref):
    acc = jnp.dot(_mean(parts_ref, cnt_ref), wl1_ref[...],
                  preferred_element_type=jnp.float32)
    acc = acc + jnp.dot(x_ref[...], wr1_ref[...],
                        preferred_element_type=jnp.float32)
    h = jnp.maximum(acc + b_ref[...], 0.0)
    z_ref[...] = jnp.dot(h, wl2_ref[...], preferred_element_type=jnp.float32)
    r2_ref[...] = jnp.dot(h, wr2_ref[...], preferred_element_type=jnp.float32)


def _t12(parts, cnt, x1k, W_l1, W_r1, b1, W_l2, W_r2):
    M, H = 1024, 1024
    BM = 512
    return pl.pallas_call(
        _t12_body,
        grid=(M // BM,),
        in_specs=[
            pl.BlockSpec((NC, BM, D), lambda i: (0, i, 0)),
            pl.BlockSpec((NC, BM, CW), lambda i: (0, i, 0)),
            pl.BlockSpec((BM, D), lambda i: (i, 0)),
            pl.BlockSpec((D, H), lambda i: (0, 0)),
            pl.BlockSpec((D, H), lambda i: (0, 0)),
            pl.BlockSpec((1, H), lambda i: (0, 0)),
            pl.BlockSpec((H, D), lambda i: (0, 0)),
            pl.BlockSpec((H, D), lambda i: (0, 0)),
        ],
        out_specs=[
            pl.BlockSpec((BM, D), lambda i: (i, 0)),
            pl.BlockSpec((BM, D), lambda i: (i, 0)),
        ],
        out_shape=[
            jax.ShapeDtypeStruct((M, D), jnp.float32),
            jax.ShapeDtypeStruct((M, D), jnp.float32),
        ],
    )(parts, cnt, x1k, W_l1, W_r1, b1.reshape(1, H), W_l2, W_r2)


def _t3_body(parts_ref, cnt_ref, r2_ref, b_ref, out_ref):
    o = _mean(parts_ref, cnt_ref) + r2_ref[...] + b_ref[...]
    m = jnp.max(o, axis=1, keepdims=True)
    e = jnp.exp(o - m)
    lse = jnp.log(jnp.sum(e, axis=1, keepdims=True))
    out_ref[...] = (o - m) - lse


def _t3(parts, cnt, r2, b):
    M = 1024
    return pl.pallas_call(
        _t3_body,
        grid=(1,),
        in_specs=[
            pl.BlockSpec((NC, M, D), lambda i: (0, 0, 0)),
            pl.BlockSpec((NC, M, CW), lambda i: (0, 0, 0)),
            pl.BlockSpec((M, D), lambda i: (0, 0)),
            pl.BlockSpec((1, D), lambda i: (0, 0)),
        ],
        out_specs=pl.BlockSpec((M, D), lambda i: (0, 0)),
        out_shape=jax.ShapeDtypeStruct((M, D), jnp.float32),
    )(parts, cnt, r2, b.reshape(1, D))


def kernel(x, edge_index1, edge_index2, W_l1, W_r1, b1, W_l2, W_r2, b2,
           n_dst1, n_dst2):
    off1 = n_dst1 - 4000
    x4k = lax.dynamic_slice_in_dim(x, off1, 4000, axis=0)
    x1k = x4k[:1024]
    zrow = jnp.zeros((16, D), jnp.float32)
    zcnt = jnp.zeros((16, CW), jnp.float32)
    onesrow = jnp.ones((K, CW), jnp.float32)

    src1, dst1 = _pad_edges(edge_index1, 160000, 163840, 4000)
    parts1, cnt1 = _make_segsum(P=4000, Ep=163840, n_keep=1024, n_acc=4096)(
        x4k, src1, dst1, zrow, zcnt, onesrow)
    z, r2 = _t12(parts1, cnt1, x1k, W_l1, W_r1, b1, W_l2, W_r2)

    src2, dst2 = _pad_edges(edge_index2, 64000, 65536, 1024)
    parts2, cnt2 = _make_segsum(P=1024, Ep=65536, n_keep=1024, n_acc=1280)(
        z, src2, dst2, zrow, zcnt, onesrow)
    return _t3(parts2, cnt2, r2, b2)


# speedup vs baseline: 1.1042x; 1.0206x over previous
import functools

import jax
import jax.numpy as jnp
from jax import lax
from jax.experimental import pallas as pl
from jax.experimental.pallas import tpu as pltpu
from jax.experimental.pallas import tpu_sc as plsc

NC = 2
NS = 16
NW = NC * NS
D = 256
CW = 16
K = 64


@functools.lru_cache(maxsize=None)
def _make_segsum(P, Ep, n_keep, n_acc, Kc=K, spmem_table=False):
    per_w = Ep // NW
    assert per_w * NW == Ep and per_w % Kc == 0 and per_w % 16 == 0
    ncap = per_w + 2 * Kc
    assert n_acc >= n_keep + 16 and n_acc % 8 == 0
    rps = (-(-n_keep // NS) + 7) // 8 * 8
    n_full = n_keep // rps
    rem = n_keep - n_full * rps
    zps = (-(-n_acc // NS) + 7) // 8 * 8
    z_full = n_acc // zps
    z_rem = n_acc - z_full * zps
    B0 = 16
    assert rps % B0 == 0 and rem % B0 == 0 and zps % B0 == 0 and z_rem % B0 == 0
    mesh = plsc.VectorSubcoreMesh(core_axis_name="c", subcore_axis_name="s",
                                  num_cores=NC, num_subcores=NS)

    @functools.partial(
        pl.kernel,
        out_type=(jax.ShapeDtypeStruct((NC, n_keep, D), jnp.float32),
                  jax.ShapeDtypeStruct((NC, n_keep, CW), jnp.float32)),
        mesh=mesh,
        scratch_types=[
            pltpu.VMEM_SHARED((n_acc, D), jnp.float32),
            pltpu.VMEM_SHARED((n_acc, CW), jnp.float32),
            pltpu.VMEM((per_w,), jnp.int32),
            pltpu.VMEM((per_w,), jnp.int32),
            pltpu.VMEM((ncap,), jnp.int32),
            pltpu.VMEM((ncap,), jnp.int32),
            pltpu.VMEM((2, Kc, D), jnp.float32),
            pltpu.VMEM((Kc, CW), jnp.float32),
            pltpu.VMEM((B0, D), jnp.float32),
            pltpu.VMEM((B0, CW), jnp.float32),
            pltpu.SemaphoreType.DMA,
            pltpu.SemaphoreType.DMA,
        ] + ([pltpu.VMEM_SHARED((P, D), jnp.float32)] if spmem_table else []),
        compiler_params=pltpu.CompilerParams(use_tc_tiling_on_sc=False,
                                             needs_layout_passes=False),
    )
    def segsum(table_hbm, src_hbm, dst_hbm, zrow_hbm, zcnt_hbm, onesrow_hbm,
               parts_hbm, cparts_hbm,
               acc_sh, cacc_sh, src_v, dst_v, srcc_v, dstc_v, rows_v, ones_v,
               buf_v, bufc_v, sem0, sem1, *maybe_table_sh):
        c = lax.axis_index("c")
        s = lax.axis_index("s")
        wid = s * NC + c

        if spmem_table:
            table_sh = maybe_table_sh[0]
            tps = P // NS
            toff = pl.multiple_of(s * tps, 8)
            for j in range(tps // B0):
                pltpu.sync_copy(table_hbm.at[pl.ds(toff + j * B0, B0)], buf_v)
                pltpu.sync_copy(buf_v, table_sh.at[pl.ds(toff + j * B0, B0)])
            gather_src = table_sh
        else:
            gather_src = table_hbm

        pltpu.sync_copy(src_hbm.at[pl.ds(wid * per_w, per_w)], src_v)
        pltpu.sync_copy(dst_hbm.at[pl.ds(wid * per_w, per_w)], dst_v)
        pltpu.sync_copy(zrow_hbm, buf_v)
        pltpu.sync_copy(zcnt_hbm, bufc_v)
        pltpu.sync_copy(onesrow_hbm, ones_v)

        zoff = pl.multiple_of(s * zps, 8)

        @pl.when(s < z_full)
        def _zero_full():
            for j in range(zps // B0):
                pltpu.sync_copy(buf_v, acc_sh.at[pl.ds(zoff + j * B0, B0)])
            for j in range(zps // B0):
                pltpu.sync_copy(bufc_v, cacc_sh.at[pl.ds(zoff + j * B0, B0)])
        if z_rem:
            @pl.when(s == z_full)
            def _zero_rem():
                for j in range(z_rem // B0):
                    pltpu.sync_copy(
                        buf_v, acc_sh.at[pl.ds(z_full * zps + j * B0, B0)])
                for j in range(z_rem // B0):
                    pltpu.sync_copy(
                        bufc_v, cacc_sh.at[pl.ds(z_full * zps + j * B0, B0)])

        lane = lax.iota(jnp.int32, 16)

        def fill(g, carry):
            srcc_v[pl.ds(g * 16, 16)] = lane
            dstc_v[pl.ds(g * 16, 16)] = lane + n_keep
            return carry

        lax.fori_loop(0, ncap // 16, fill, 0)

        def compact(g, off):
            sv = src_v[pl.ds(g * 16, 16)]
            dv = dst_v[pl.ds(g * 16, 16)]
            m = dv < n_keep
            packed = jnp.bitwise_or(jnp.left_shift(sv, 12), dv)
            _, pv = plsc.sort_key_val(dv, packed)
            nk = plsc.all_reduce_population_count(m)
            keep = lane < nk
            pos = off + lane
            plsc.store_scatter(srcc_v, [pos], jnp.right_shift(pv, 12), mask=keep)
            plsc.store_scatter(dstc_v, [pos], jnp.bitwise_and(pv, 4095), mask=keep)
            return off + nk

        offv = lax.fori_loop(0, per_w // 16, compact,
                             jnp.zeros((16,), jnp.int32))
        n_edges = jnp.max(offv)
        n_pairs = (n_edges + 2 * Kc - 1) // (2 * Kc)

        plsc.subcore_barrier()

        sems = (sem0, sem1)

        def gather(ci, b):
            pltpu.async_copy(gather_src.at[srcc_v.at[pl.ds(ci * Kc, Kc)]],
                             rows_v.at[b], sems[b])

        def gwait(ci, b):
            pltpu.make_async_copy(gather_src.at[srcc_v.at[pl.ds(ci * Kc, Kc)]],
                                  rows_v.at[b], sems[b]).wait()

        def scatter(ci, b):
            idx = dstc_v.at[pl.ds(ci * Kc, Kc)]
            pltpu.sync_copy(rows_v.at[b], acc_sh.at[idx], add=True)
            pltpu.sync_copy(ones_v, cacc_sh.at[idx], add=True)

        @pl.when(n_pairs > 0)
        def _prologue():
            gather(0, 0)

        def pair(i, carry):
            c0 = i * 2
            gwait(c0, 0)
            gather(c0 + 1, 1)
            scatter(c0, 0)
            gwait(c0 + 1, 1)

            @pl.when(i < n_pairs - 1)
            def _next():
                gather(c0 + 2, 0)
            scatter(c0 + 1, 1)
            return carry

        lax.fori_loop(0, n_pairs, pair, 0)
        plsc.subcore_barrier()

        row_off = pl.multiple_of(s * rps, 8)

        @pl.when(s < n_full)
        def _out_full():
            for j in range(rps // B0):
                pltpu.sync_copy(acc_sh.at[pl.ds(row_off + j * B0, B0)], buf_v)
                pltpu.sync_copy(buf_v,
                                parts_hbm.at[c, pl.ds(row_off + j * B0, B0)])
            for j in range(rps // B0):
                pltpu.sync_copy(cacc_sh.at[pl.ds(row_off + j * B0, B0)], bufc_v)
                pltpu.sync_copy(bufc_v,
                                cparts_hbm.at[c, pl.ds(row_off + j * B0, B0)])
        if rem:
            @pl.when(s == n_full)
            def _out_rem():
                for j in range(rem // B0):
                    off_r = n_full * rps + j * B0
                    pltpu.sync_copy(acc_sh.at[pl.ds(off_r, B0)], buf_v)
                    pltpu.sync_copy(buf_v, parts_hbm.at[c, pl.ds(off_r, B0)])
                for j in range(rem // B0):
                    off_r = n_full * rps + j * B0
                    pltpu.sync_copy(cacc_sh.at[pl.ds(off_r, B0)], bufc_v)
                    pltpu.sync_copy(bufc_v, cparts_hbm.at[c, pl.ds(off_r, B0)])

    return segsum


def _pad_edges(edge_index, E, Ep, dump):
    pad = Ep - E
    spread = jnp.arange(pad, dtype=jnp.int32) % K
    src = jnp.concatenate([edge_index[0], spread])
    dst = jnp.concatenate([edge_index[1], dump + spread])
    return src, dst


def _mean(parts_ref, cnt_ref):
    s = parts_ref[0] + parts_ref[1]
    cnt = cnt_ref[0, :, :1] + cnt_ref[1, :, :1]
    return s / jnp.maximum(cnt, 1.0)


def _t12_body(parts_ref, cnt_ref, x_ref, wl1_ref, wr1_ref, b_ref,
              wl2_ref, wr2_ref, z_ref, r2_ref):
    acc = jnp.dot(_mean(parts_ref, cnt_ref), wl1_ref[...],
                  preferred_element_type=jnp.float32)
    acc = acc + jnp.dot(x_ref[...], wr1_ref[...],
                        preferred_element_type=jnp.float32)
    h = jnp.maximum(acc + b_ref[...], 0.0)
    z_ref[...] = jnp.dot(h, wl2_ref[...], preferred_element_type=jnp.float32)
    r2_ref[...] = jnp.dot(h, wr2_ref[...], preferred_element_type=jnp.float32)


def _t12(parts, cnt, x1k, W_l1, W_r1, b1, W_l2, W_r2):
    M, H = 1024, 1024
    BM = 512
    return pl.pallas_call(
        _t12_body,
        grid=(M // BM,),
        in_specs=[
            pl.BlockSpec((NC, BM, D), lambda i: (0, i, 0)),
            pl.BlockSpec((NC, BM, CW), lambda i: (0, i, 0)),
            pl.BlockSpec((BM, D), lambda i: (i, 0)),
            pl.BlockSpec((D, H), lambda i: (0, 0)),
            pl.BlockSpec((D, H), lambda i: (0, 0)),
            pl.BlockSpec((1, H), lambda i: (0, 0)),
            pl.BlockSpec((H, D), lambda i: (0, 0)),
            pl.BlockSpec((H, D), lambda i: (0, 0)),
        ],
        out_specs=[
            pl.BlockSpec((BM, D), lambda i: (i, 0)),
            pl.BlockSpec((BM, D), lambda i: (i, 0)),
        ],
        out_shape=[
            jax.ShapeDtypeStruct((M, D), jnp.float32),
            jax.ShapeDtypeStruct((M, D), jnp.float32),
        ],
    )(parts, cnt, x1k, W_l1, W_r1, b1.reshape(1, H), W_l2, W_r2)


def _t3_body(parts_ref, cnt_ref, r2_ref, b_ref, out_ref):
    o = _mean(parts_ref, cnt_ref) + r2_ref[...] + b_ref[...]
    m = jnp.max(o, axis=1, keepdims=True)
    e = jnp.exp(o - m)
    lse = jnp.log(jnp.sum(e, axis=1, keepdims=True))
    out_ref[...] = (o - m) - lse


def _t3(parts, cnt, r2, b):
    M = 1024
    return pl.pallas_call(
        _t3_body,
        grid=(1,),
        in_specs=[
            pl.BlockSpec((NC, M, D), lambda i: (0, 0, 0)),
            pl.BlockSpec((NC, M, CW), lambda i: (0, 0, 0)),
            pl.BlockSpec((M, D), lambda i: (0, 0)),
            pl.BlockSpec((1, D), lambda i: (0, 0)),
        ],
        out_specs=pl.BlockSpec((M, D), lambda i: (0, 0)),
        out_shape=jax.ShapeDtypeStruct((M, D), jnp.float32),
    )(parts, cnt, r2, b.reshape(1, D))


def kernel(x, edge_index1, edge_index2, W_l1, W_r1, b1, W_l2, W_r2, b2,
           n_dst1, n_dst2):
    off1 = n_dst1 - 4000
    x4k = lax.dynamic_slice_in_dim(x, off1, 4000, axis=0)
    x1k = x4k[:1024]
    zrow = jnp.zeros((16, D), jnp.float32)
    zcnt = jnp.zeros((16, CW), jnp.float32)
    onesrow = jnp.ones((128, CW), jnp.float32)

    src1, dst1 = _pad_edges(edge_index1, 160000, 163840, 4000)
    parts1, cnt1 = _make_segsum(P=4000, Ep=163840, n_keep=1024, n_acc=1280, Kc=128)(
        x4k, src1, dst1, zrow, zcnt, onesrow)
    z, r2 = _t12(parts1, cnt1, x1k, W_l1, W_r1, b1, W_l2, W_r2)

    src2, dst2 = _pad_edges(edge_index2, 64000, 65536, 1024)
    parts2, cnt2 = _make_segsum(P=1024, Ep=65536, n_keep=1024, n_acc=1280, Kc=128)(
        z, src2, dst2, zrow, zcnt, onesrow)
    return _t3(parts2, cnt2, r2, b2)
